# baseline (speedup 1.0000x reference)
import functools

import jax
import jax.numpy as jnp
from jax import lax
from jax.experimental import pallas as pl
from jax.experimental.pallas import tpu as pltpu
from jax.experimental.pallas import tpu_sc as plsc

_N = 10000
_E = 320000
_IN = 128
_H = 64
_OUT = 128
_EF = 16

_NW = 32
_CHUNK = 128
_E_PAD = 327680
_WROW = 128
_N_ACC = 10240
_TOTC = _E_PAD // _CHUNK
_TOTC_0 = 1792
_TOTC_1 = _TOTC - _TOTC_0
_E_0 = _TOTC_0 * _CHUNK
_E_1 = _TOTC_1 * _CHUNK
_TOTC_PAD = _TOTC + 64

_f32 = jnp.float32
_bf16 = jnp.bfloat16

_MESH2 = dict(core_axis_name="c", subcore_axis_name="s")


def _lrelu(x):
    return jnp.where(x >= 0, x, 0.01 * x)


def _dotT(a, w):
    return lax.dot_general(a, w, (((1,), (1,)), ((), ())),
                           preferred_element_type=_f32)


def _dotTb(a, w):
    return lax.dot_general(a.astype(_bf16), w, (((1,), (1,)), ((), ())),
                           preferred_element_type=_f32)


def _pre_body(nf_ref, x_ref, w1s_ref, w1d_ref, b1_ref, s_ref, d_ref):
    nf = nf_ref[...]
    x = x_ref[...]
    ps = _dotT(nf, w1s_ref[...]) + b1_ref[...]
    pd = _dotT(nf, w1d_ref[...])
    nrow = nf.shape[0]
    z2 = jnp.zeros((nrow, 2), _f32)
    z60 = jnp.zeros((nrow, _WROW - _H - 4), _f32)
    s_ref[...] = jnp.concatenate([ps, x, z2, z60], axis=1)
    d_ref[...] = jnp.concatenate([pd, z2, x, z60], axis=1)


def _pre(nf, x, w1s, w1d, b1):
    blk = 1000
    grid = (_N // blk,)
    return pl.pallas_call(
        _pre_body,
        grid=grid,
        in_specs=[
            pl.BlockSpec((blk, _IN), lambda i: (i, 0)),
            pl.BlockSpec((blk, 2), lambda i: (i, 0)),
            pl.BlockSpec((_H, _IN), lambda i: (0, 0)),
            pl.BlockSpec((_H, _IN), lambda i: (0, 0)),
            pl.BlockSpec((1, _H), lambda i: (0, 0)),
        ],
        out_specs=[
            pl.BlockSpec((blk, _WROW), lambda i: (i, 0)),
            pl.BlockSpec((blk, _WROW), lambda i: (i, 0)),
        ],
        out_shape=[
            jax.ShapeDtypeStruct((_N, _WROW), _f32),
            jax.ShapeDtypeStruct((_N, _WROW), _f32),
        ],
    )(nf, x, w1s, w1d, b1)


def _gather(tab_s, tab_d, srcc, dstc, hb, nchunks, nc0, nc1):
    mesh = plsc.VectorSubcoreMesh(**_MESH2)
    NP = 2
    nstage = max(nc0, nc1)

    @functools.partial(
        pl.kernel,
        mesh=mesh,
        out_type=jax.ShapeDtypeStruct((nchunks * _CHUNK, _WROW), _f32),
        scratch_types=[
            pltpu.VMEM((nstage, _CHUNK), jnp.int32),
            pltpu.VMEM((nstage, _CHUNK), jnp.int32),
            pltpu.VMEM((2 * NP, _CHUNK, _WROW), _f32),
            pltpu.SemaphoreType.DMA((2 * NP,)),
            pltpu.SemaphoreType.DMA((NP,)),
        ],
    )
    def k(s_hbm, d_hbm, src_hbm, dst_hbm, out_hbm,
          idx_s, idx_d, bufs, g_sems, w_sems):
        c = lax.axis_index("c")
        s = lax.axis_index("s")
        ncw = jnp.where(c == 0, nc0, nc1)
        chunk0 = pl.multiple_of(
            jnp.where(c == 0, s * nc0, 16 * nc0 + s * nc1), 8)
        pltpu.sync_copy(src_hbm.at[pl.ds(hb + chunk0, nstage)], idx_s)
        pltpu.sync_copy(dst_hbm.at[pl.ds(hb + chunk0, nstage)], idx_d)

        def gather_desc(p, jj, is_d):
            b = 2 * p + is_d
            tab = d_hbm if is_d else s_hbm
            idx = idx_d if is_d else idx_s
            return pltpu.make_async_copy(tab.at[idx.at[jj]], bufs.at[b],
                                         g_sems.at[b])

        def out_slice(jj):
            off = pl.multiple_of((chunk0 + jj) * _CHUNK, _CHUNK)
            return out_hbm.at[pl.ds(off, _CHUNK)]

        def add_pair(p):
            bs, bd = 2 * p, 2 * p + 1

            def rowadd(r, carry):
                for kk in range(_WROW // 16):
                    sl = pl.ds(kk * 16, 16)
                    bufs[bs, r, sl] = bufs[bs, r, sl] + bufs[bd, r, sl]
                return carry

            lax.fori_loop(0, _CHUNK, rowadd, 0)

        for p in range(NP):
            gather_desc(p, p, 0).start()
            gather_desc(p, p, 1).start()

        def group(g, carry):
            for p in range(NP):
                jj = g * NP + p
                gather_desc(p, jj, 0).wait()
                gather_desc(p, jj, 1).wait()
                add_pair(p)
                dst_sl = out_slice(jj)
                pltpu.async_copy(bufs.at[2 * p], dst_sl, w_sems.at[p])
                njj = jj + NP

                @pl.when(njj < ncw)
                def _():
                    pltpu.make_async_copy(bufs.at[2 * p], dst_sl,
                                          w_sems.at[p]).wait()
                    gather_desc(p, njj, 0).start()
                    gather_desc(p, njj, 1).start()
            return carry

        lax.fori_loop(0, ncw // NP, group, 0)
        for p in range(NP):
            jj = ncw - NP + p
            pltpu.make_async_copy(bufs.at[2 * p], out_slice(jj),
                                  w_sems.at[p]).wait()

    return k(tab_s, tab_d, srcc, dstc)


def _edge_body(g_ref, ea_ref, w1e_ref, w1r_ref,
               w2_ref, b2_ref, w3_ref, b3_ref, w4_ref, b4_ref, w5_ref, b5_ref,
               wc1_ref, bc1_ref, wc2_ref, bc2_ref, wc3_ref, bc3_ref,
               wc4_ref, bc4_ref, wc5_ref, out_ref):
    gs = g_ref[...]
    dxy = gs[:, _H:_H + 2] - gs[:, _H + 2:_H + 4]
    radial = dxy[:, 0:1] * dxy[:, 0:1] + dxy[:, 1:2] * dxy[:, 1:2]
    inv = 1.0 / (jnp.sqrt(radial) + 1e-30)
    z = (gs[:, 0:_H] + radial * w1r_ref[...]
         + _dotTb(ea_ref[...], w1e_ref[...]))
    a = _lrelu(z)
    a = _lrelu(_dotTb(a, w2_ref[...]) + b2_ref[...])
    a = _lrelu(_dotTb(a, w3_ref[...]) + b3_ref[...])
    a = _lrelu(_dotTb(a, w4_ref[...]) + b4_ref[...])
    msg_h = _dotTb(a, w5_ref[...]) + b5_ref[...]
    c = _lrelu(_dotTb(msg_h, wc1_ref[...]) + bc1_ref[...])
    c = _lrelu(_dotTb(c, wc2_ref[...]) + bc2_ref[...])
    c = _lrelu(_dotTb(c, wc3_ref[...]) + bc3_ref[...])
    c = _lrelu(_dotTb(c, wc4_ref[...]) + bc4_ref[...])
    cc = _dotT(c, wc5_ref[...])
    mx = cc * (dxy * inv)
    nrow = gs.shape[0]
    ones = jnp.ones((nrow, 1), _f32)
    zer = jnp.zeros((nrow, _WROW - _H - 3), _f32)
    out_ref[...] = jnp.concatenate([msg_h, mx, ones, zer], axis=1)


def _edge(g, ea, wts, blk_off):
    blk = 2048
    nb = g.shape[0] // blk
    nea = _E // blk - 1
    grid = (nb,)
    off = blk_off
    full = lambda shp: pl.BlockSpec(shp, lambda i: tuple(0 for _ in shp))
    w_specs = [full(w.shape) for w in wts]
    return pl.pallas_call(
        _edge_body,
        grid=grid,
        in_specs=[
            pl.BlockSpec((blk, _WROW), lambda i: (i, 0)),
            pl.BlockSpec((blk, _EF), lambda i: (jnp.minimum(i + off, nea), 0)),
        ] + w_specs,
        out_specs=pl.BlockSpec((blk, _WROW), lambda i: (i, 0)),
        out_shape=jax.ShapeDtypeStruct((g.shape[0], _WROW), _f32),
    )(g, ea, *wts)


def _scatter(msg, dstc_s, hb):
    mesh = plsc.VectorSubcoreMesh(**_MESH2)
    rows_per_s = _N_ACC // 16
    NB = 2
    nchunks = msg.shape[0] // _CHUNK
    ncw = nchunks // _NW
    NG = ncw // NB

    @functools.partial(
        pl.kernel,
        mesh=mesh,
        out_type=jax.ShapeDtypeStruct((2, _N_ACC, _WROW), _f32),
        scratch_types=[
            pltpu.VMEM_SHARED((_N_ACC, _WROW), _f32),
            pltpu.VMEM((ncw, _CHUNK), jnp.int32),
            pltpu.VMEM((NB, _CHUNK, _WROW), _f32),
            pltpu.SemaphoreType.DMA((NB,)),
            pltpu.SemaphoreType.DMA((NB,)),
        ],
    )
    def k(msg_hbm, dst_hbm, out_hbm, acc_sh, idx_d, bufs,
          g_sems, w_sems):
        c = lax.axis_index("c")
        s = lax.axis_index("s")
        r0 = pl.multiple_of(s * rows_per_s, 8)

        def zrow(r, carry):
            for kk in range(_WROW // 16):
                bufs[0, r, pl.ds(kk * 16, 16)] = jnp.zeros((16,), _f32)
            return carry

        lax.fori_loop(0, _CHUNK, zrow, 0)
        for rr in range(rows_per_s // _CHUNK):
            pltpu.sync_copy(bufs.at[0],
                            acc_sh.at[pl.ds(r0 + rr * _CHUNK, _CHUNK)])
        wid = s * 2 + c
        chunk0 = pl.multiple_of(wid * ncw, 8)
        base = chunk0 * _CHUNK
        pltpu.sync_copy(dst_hbm.at[pl.ds(hb + chunk0, ncw)], idx_d)
        plsc.subcore_barrier()

        def load_desc(b, jj):
            return pltpu.make_async_copy(
                msg_hbm.at[pl.ds(base + jj * _CHUNK, _CHUNK)], bufs.at[b],
                g_sems.at[b])

        for b in range(NB):
            load_desc(b, b).start()

        def group(g, carry):
            for b in range(NB):
                jj = g * NB + b
                load_desc(b, jj).wait()
                acc_sl = acc_sh.at[idx_d.at[jj]]
                pltpu.async_copy(bufs.at[b], acc_sl, w_sems.at[b], add=True)
                njj = jj + NB

                @pl.when(njj < ncw)
                def _():
                    pltpu.make_async_copy(bufs.at[b], acc_sl,
                                          w_sems.at[b]).wait()
                    load_desc(b, njj).start()
            return carry

        lax.fori_loop(0, NG, group, 0)
        for b in range(NB):
            jj = (NG - 1) * NB + b
            pltpu.make_async_copy(bufs.at[b], acc_sh.at[idx_d.at[jj]],
                                  w_sems.at[b]).wait()
        plsc.subcore_barrier()
        pltpu.sync_copy(acc_sh.at[pl.ds(r0, rows_per_s)],
                        out_hbm.at[c, pl.ds(r0, rows_per_s)])

    return k(msg, dstc_s)


def _post_body(nf_ref, p00_ref, p01_ref, p10_ref, p11_ref,
               wv1_ref, bv1_ref, wv2_ref, bv2_ref, wv3_ref, bv3_ref,
               wv4_ref, bv4_ref, wv5_ref,
               wn1_ref, bn1_ref, wn2_ref, bn2_ref, wn3_ref, bn3_ref,
               wn4_ref, bn4_ref, wn5_ref, bn5_ref, wl_ref, bl_ref, out_ref):
    nf = nf_ref[...]
    acc = (p00_ref[0] + p01_ref[0]) + (p10_ref[0] + p11_ref[0])
    h_neigh = acc[:, 0:_H]
    sx = acc[:, _H:_H + 2]
    cnt = acc[:, _H + 2:_H + 3]
    xn = sx / jnp.maximum(cnt, 1.0)
    v = _lrelu(_dotT(nf, wv1_ref[...]) + bv1_ref[...])
    v = _lrelu(_dotT(v, wv2_ref[...]) + bv2_ref[...])
    v = _lrelu(_dotT(v, wv3_ref[...]) + bv3_ref[...])
    v = _lrelu(_dotT(v, wv4_ref[...]) + bv4_ref[...])
    vout = _dotT(v, wv5_ref[...])
    xn = xn + vout * nf[:, 0:2]
    a = jnp.concatenate([nf, h_neigh], axis=1)
    a = _lrelu(_dotT(a, wn1_ref[...]) + bn1_ref[...])
    a = _lrelu(_dotT(a, wn2_ref[...]) + bn2_ref[...])
    a = _lrelu(_dotT(a, wn3_ref[...]) + bn3_ref[...])
    a = _lrelu(_dotT(a, wn4_ref[...]) + bn4_ref[...])
    a = _dotT(a, wn5_ref[...]) + bn5_ref[...]
    h = _dotT(a, wl_ref[...]) + bl_ref[...]
    out_ref[...] = jnp.concatenate([xn, h], axis=1)


def _post(nf, p0, p1, wts):
    blk = 1000
    grid = (_N // blk,)
    full = lambda shp: pl.BlockSpec(shp, lambda i: tuple(0 for _ in shp))
    w_specs = [full(w.shape) for w in wts]
    return pl.pallas_call(
        _post_body,
        grid=grid,
        in_specs=[
            pl.BlockSpec((blk, _IN), lambda i: (i, 0)),
            pl.BlockSpec((1, blk, _WROW), lambda i: (0, i, 0)),
            pl.BlockSpec((1, blk, _WROW), lambda i: (1, i, 0)),
            pl.BlockSpec((1, blk, _WROW), lambda i: (0, i, 0)),
            pl.BlockSpec((1, blk, _WROW), lambda i: (1, i, 0)),
        ] + w_specs,
        out_specs=pl.BlockSpec((blk, 2 + _OUT), lambda i: (i, 0)),
        out_shape=jax.ShapeDtypeStruct((_N, 2 + _OUT), _f32),
    )(nf, p0, p0, p1, p1, *wts)


def kernel(node_feat, feat, edge_index, edge_attr, params):
    w1, b1 = params["edge_mlp"][0]
    w1s = w1[:, :_IN]
    w1d = w1[:, _IN:2 * _IN]
    w1r = w1[:, 2 * _IN:2 * _IN + 1].T
    w1e = w1[:, 2 * _IN + 1:].astype(_bf16)
    b1r = b1.reshape(1, _H)

    def row(b):
        return b.reshape(1, -1)

    em = params["edge_mlp"]
    cm = params["coord_mlp"]
    edge_wts = [w1e, w1r]
    for i in (1, 2, 3, 4):
        edge_wts += [em[i][0].astype(_bf16), row(em[i][1])]
    for i in (0, 1, 2, 3):
        edge_wts += [cm[i][0].astype(_bf16), row(cm[i][1])]
    edge_wts.append(cm[4][0])

    vm = params["vel_mlp"]
    nm = params["node_mlp"]
    node_wts = []
    for i in (0, 1, 2, 3):
        node_wts += [vm[i][0], row(vm[i][1])]
    node_wts.append(vm[4][0])
    for i in range(5):
        node_wts += [nm[i][0], row(nm[i][1])]
    node_wts += [params["linh"][0], row(params["linh"][1])]

    src = edge_index[0]
    dst = edge_index[1]
    pad = _E_PAD - _E
    padc = (_TOTC_PAD - _TOTC) * _CHUNK
    src_p = jnp.concatenate([src, jnp.zeros((pad + padc,), jnp.int32)])
    dst_g = jnp.concatenate([dst, jnp.zeros((pad + padc,), jnp.int32)])
    dst_s = jnp.concatenate([dst, jnp.full((pad,), _N, jnp.int32)])
    srcc = src_p.reshape(_TOTC_PAD, _CHUNK)
    dstc = dst_g.reshape(_TOTC_PAD, _CHUNK)
    dstsc = dst_s.reshape(_TOTC, _CHUNK)

    tab_s, tab_d = _pre(node_feat, feat, w1s, w1d, b1r)
    g0 = _gather(tab_s, tab_d, srcc, dstc, 0, _TOTC_0, 96, 16)
    g1 = _gather(tab_s, tab_d, srcc, dstc, _TOTC_0, _TOTC_1, 8, 40)
    msg0 = _edge(g0, edge_attr, edge_wts, 0)
    msg1 = _edge(g1, edge_attr, edge_wts, _E_0 // 2048)
    p0 = _scatter(msg0, dstsc, 0)
    p1 = _scatter(msg1, dstsc, _TOTC_0)
    out = _post(node_feat, p0, p1, node_wts)
    return out


# speedup vs baseline: 1.0086x; 1.0086x over previous
import functools

import jax
import jax.numpy as jnp
from jax import lax
from jax.experimental import pallas as pl
from jax.experimental.pallas import tpu as pltpu
from jax.experimental.pallas import tpu_sc as plsc

_N = 10000
_E = 320000
_IN = 128
_H = 64
_OUT = 128
_EF = 16

_NW = 32
_CHUNK = 128
_E_PAD = 327680
_WROW = 128
_N_ACC = 10240
_TOTC = _E_PAD // _CHUNK
_TOTC_0 = 1792
_TOTC_1 = _TOTC - _TOTC_0
_E_0 = _TOTC_0 * _CHUNK
_E_1 = _TOTC_1 * _CHUNK
_TOTC_PAD = _TOTC + 64

_f32 = jnp.float32
_bf16 = jnp.bfloat16

_MESH2 = dict(core_axis_name="c", subcore_axis_name="s")


def _lrelu(x):
    return jnp.where(x >= 0, x, 0.01 * x)


def _dotT(a, w):
    return lax.dot_general(a, w, (((1,), (1,)), ((), ())),
                           preferred_element_type=_f32)


def _dotTb(a, w):
    return lax.dot_general(a.astype(_bf16), w, (((1,), (1,)), ((), ())),
                           preferred_element_type=_f32)


def _pre_body(nf_ref, x_ref, w1s_ref, w1d_ref, b1_ref, s_ref, d_ref):
    nf = nf_ref[...]
    x = x_ref[...]
    ps = _dotT(nf, w1s_ref[...]) + b1_ref[...]
    pd = _dotT(nf, w1d_ref[...])
    nrow = nf.shape[0]
    z2 = jnp.zeros((nrow, 2), _f32)
    z60 = jnp.zeros((nrow, _WROW - _H - 4), _f32)
    s_ref[...] = jnp.concatenate([ps, x, z2, z60], axis=1)
    d_ref[...] = jnp.concatenate([pd, z2, x, z60], axis=1)


def _pre(nf, x, w1s, w1d, b1):
    blk = 1000
    grid = (_N // blk,)
    return pl.pallas_call(
        _pre_body,
        grid=grid,
        in_specs=[
            pl.BlockSpec((blk, _IN), lambda i: (i, 0)),
            pl.BlockSpec((blk, 2), lambda i: (i, 0)),
            pl.BlockSpec((_H, _IN), lambda i: (0, 0)),
            pl.BlockSpec((_H, _IN), lambda i: (0, 0)),
            pl.BlockSpec((1, _H), lambda i: (0, 0)),
        ],
        out_specs=[
            pl.BlockSpec((blk, _WROW), lambda i: (i, 0)),
            pl.BlockSpec((blk, _WROW), lambda i: (i, 0)),
        ],
        out_shape=[
            jax.ShapeDtypeStruct((_N, _WROW), _f32),
            jax.ShapeDtypeStruct((_N, _WROW), _f32),
        ],
    )(nf, x, w1s, w1d, b1)


def _gather(tab_s, tab_d, srcc, dstc, hb, nchunks, nc0, nc1):
    mesh = plsc.VectorSubcoreMesh(**_MESH2)
    NP = 2
    nstage = max(nc0, nc1)

    @functools.partial(
        pl.kernel,
        mesh=mesh,
        out_type=jax.ShapeDtypeStruct((nchunks * _CHUNK, _WROW), _f32),
        scratch_types=[
            pltpu.VMEM((nstage, _CHUNK), jnp.int32),
            pltpu.VMEM((nstage, _CHUNK), jnp.int32),
            pltpu.VMEM((2 * NP, _CHUNK, _WROW), _f32),
            pltpu.SemaphoreType.DMA((2 * NP,)),
            pltpu.SemaphoreType.DMA((NP,)),
        ],
    )
    def k(s_hbm, d_hbm, src_hbm, dst_hbm, out_hbm,
          idx_s, idx_d, bufs, g_sems, w_sems):
        c = lax.axis_index("c")
        s = lax.axis_index("s")
        ncw = jnp.where(c == 0, nc0, nc1)
        chunk0 = pl.multiple_of(
            jnp.where(c == 0, s * nc0, 16 * nc0 + s * nc1), 8)
        pltpu.sync_copy(src_hbm.at[pl.ds(hb + chunk0, nstage)], idx_s)
        pltpu.sync_copy(dst_hbm.at[pl.ds(hb + chunk0, nstage)], idx_d)

        def gather_desc(p, jj, is_d):
            b = 2 * p + is_d
            tab = d_hbm if is_d else s_hbm
            idx = idx_d if is_d else idx_s
            return pltpu.make_async_copy(tab.at[idx.at[jj]], bufs.at[b],
                                         g_sems.at[b])

        def out_slice(jj):
            off = pl.multiple_of((chunk0 + jj) * _CHUNK, _CHUNK)
            return out_hbm.at[pl.ds(off, _CHUNK)]

        def add_pair(p):
            bs, bd = 2 * p, 2 * p + 1

            def rowadd(r, carry):
                for kk in range(_WROW // 16):
                    sl = pl.ds(kk * 16, 16)
                    bufs[bs, r, sl] = bufs[bs, r, sl] + bufs[bd, r, sl]
                return carry

            lax.fori_loop(0, _CHUNK, rowadd, 0)

        for p in range(NP):
            gather_desc(p, p, 0).start()
            gather_desc(p, p, 1).start()

        def group(g, carry):
            for p in range(NP):
                jj = g * NP + p
                gather_desc(p, jj, 0).wait()
                gather_desc(p, jj, 1).wait()
                add_pair(p)
                dst_sl = out_slice(jj)
                pltpu.async_copy(bufs.at[2 * p], dst_sl, w_sems.at[p])
                njj = jj + NP

                @pl.when(njj < ncw)
                def _():
                    pltpu.make_async_copy(bufs.at[2 * p], dst_sl,
                                          w_sems.at[p]).wait()
                    gather_desc(p, njj, 0).start()
                    gather_desc(p, njj, 1).start()
            return carry

        lax.fori_loop(0, ncw // NP, group, 0)
        for p in range(NP):
            jj = ncw - NP + p
            pltpu.make_async_copy(bufs.at[2 * p], out_slice(jj),
                                  w_sems.at[p]).wait()

    return k(tab_s, tab_d, srcc, dstc)


def _edge_body(g_ref, ea_ref, w1e_ref, w1r_ref,
               w2_ref, b2_ref, w3_ref, b3_ref, w4_ref, b4_ref, w5_ref, b5_ref,
               wc1_ref, bc1_ref, wc2_ref, bc2_ref, wc3_ref, bc3_ref,
               wc4_ref, bc4_ref, wc5_ref, out_ref):
    gs = g_ref[...]
    dxy = gs[:, _H:_H + 2] - gs[:, _H + 2:_H + 4]
    radial = dxy[:, 0:1] * dxy[:, 0:1] + dxy[:, 1:2] * dxy[:, 1:2]
    inv = 1.0 / (jnp.sqrt(radial) + 1e-30)
    z = (gs[:, 0:_H] + radial * w1r_ref[...]
         + _dotTb(ea_ref[...], w1e_ref[...]))
    a = _lrelu(z)
    a = _lrelu(_dotTb(a, w2_ref[...]) + b2_ref[...])
    a = _lrelu(_dotTb(a, w3_ref[...]) + b3_ref[...])
    a = _lrelu(_dotTb(a, w4_ref[...]) + b4_ref[...])
    msg_h = _dotTb(a, w5_ref[...]) + b5_ref[...]
    c = _lrelu(_dotTb(msg_h, wc1_ref[...]) + bc1_ref[...])
    c = _lrelu(_dotTb(c, wc2_ref[...]) + bc2_ref[...])
    c = _lrelu(_dotTb(c, wc3_ref[...]) + bc3_ref[...])
    c = _lrelu(_dotTb(c, wc4_ref[...]) + bc4_ref[...])
    cc = _dotT(c, wc5_ref[...])
    mx = cc * (dxy * inv)
    nrow = gs.shape[0]
    ones = jnp.ones((nrow, 1), _f32)
    zer = jnp.zeros((nrow, _WROW - _H - 3), _f32)
    out_ref[...] = jnp.concatenate([msg_h, mx, ones, zer], axis=1)


def _edge(g, ea, wts, blk_off):
    blk = 2048
    nb = g.shape[0] // blk
    nea = _E // blk - 1
    grid = (nb,)
    off = blk_off
    full = lambda shp: pl.BlockSpec(shp, lambda i: tuple(0 for _ in shp))
    w_specs = [full(w.shape) for w in wts]
    return pl.pallas_call(
        _edge_body,
        grid=grid,
        in_specs=[
            pl.BlockSpec((blk, _WROW), lambda i: (i, 0)),
            pl.BlockSpec((blk, _EF), lambda i: (jnp.minimum(i + off, nea), 0)),
        ] + w_specs,
        out_specs=pl.BlockSpec((blk, _WROW), lambda i: (i, 0)),
        out_shape=jax.ShapeDtypeStruct((g.shape[0], _WROW), _f32),
    )(g, ea, *wts)


def _scatter(msg, dstc_s, hb):
    mesh = plsc.VectorSubcoreMesh(**_MESH2)
    rows_per_s = _N_ACC // 16
    NB = 2
    nchunks = msg.shape[0] // _CHUNK
    ncw = nchunks // _NW
    NG = ncw // NB

    @functools.partial(
        pl.kernel,
        mesh=mesh,
        out_type=jax.ShapeDtypeStruct((2, _N_ACC, _WROW), _f32),
        scratch_types=[
            pltpu.VMEM_SHARED((_N_ACC, _WROW), _f32),
            pltpu.VMEM((ncw, _CHUNK), jnp.int32),
            pltpu.VMEM((NB, _CHUNK, _WROW), _f32),
            pltpu.SemaphoreType.DMA((NB,)),
            pltpu.SemaphoreType.DMA((NB,)),
        ],
    )
    def k(msg_hbm, dst_hbm, out_hbm, acc_sh, idx_d, bufs,
          g_sems, w_sems):
        c = lax.axis_index("c")
        s = lax.axis_index("s")
        r0 = pl.multiple_of(s * rows_per_s, 8)

        def zrow(r, carry):
            for kk in range(_WROW // 16):
                bufs[0, r, pl.ds(kk * 16, 16)] = jnp.zeros((16,), _f32)
            return carry

        lax.fori_loop(0, _CHUNK, zrow, 0)
        for rr in range(rows_per_s // _CHUNK):
            pltpu.sync_copy(bufs.at[0],
                            acc_sh.at[pl.ds(r0 + rr * _CHUNK, _CHUNK)])
        wid = s * 2 + c
        chunk0 = pl.multiple_of(wid * ncw, 8)
        base = chunk0 * _CHUNK
        pltpu.sync_copy(dst_hbm.at[pl.ds(hb + chunk0, ncw)], idx_d)
        plsc.subcore_barrier()

        def load_desc(b, jj):
            return pltpu.make_async_copy(
                msg_hbm.at[pl.ds(base + jj * _CHUNK, _CHUNK)], bufs.at[b],
                g_sems.at[b])

        for b in range(NB):
            load_desc(b, b).start()

        def group(g, carry):
            for b in range(NB):
                jj = g * NB + b
                load_desc(b, jj).wait()
                acc_sl = acc_sh.at[idx_d.at[jj]]
                pltpu.async_copy(bufs.at[b], acc_sl, w_sems.at[b], add=True)
                njj = jj + NB

                @pl.when(njj < ncw)
                def _():
                    pltpu.make_async_copy(bufs.at[b], acc_sl,
                                          w_sems.at[b]).wait()
                    load_desc(b, njj).start()
            return carry

        lax.fori_loop(0, NG, group, 0)
        for b in range(NB):
            jj = (NG - 1) * NB + b
            pltpu.make_async_copy(bufs.at[b], acc_sh.at[idx_d.at[jj]],
                                  w_sems.at[b]).wait()
        plsc.subcore_barrier()
        pltpu.sync_copy(acc_sh.at[pl.ds(r0, rows_per_s)],
                        out_hbm.at[c, pl.ds(r0, rows_per_s)])

    return k(msg, dstc_s)


def _post_body(nf_ref, p00_ref, p01_ref, p10_ref, p11_ref,
               wv1_ref, bv1_ref, wv2_ref, bv2_ref, wv3_ref, bv3_ref,
               wv4_ref, bv4_ref, wv5_ref,
               wn1_ref, bn1_ref, wn2_ref, bn2_ref, wn3_ref, bn3_ref,
               wn4_ref, bn4_ref, wn5_ref, bn5_ref, wl_ref, bl_ref, out_ref):
    nf = nf_ref[...]
    acc = (p00_ref[0] + p01_ref[0]) + (p10_ref[0] + p11_ref[0])
    h_neigh = acc[:, 0:_H]
    sx = acc[:, _H:_H + 2]
    cnt = acc[:, _H + 2:_H + 3]
    xn = sx / jnp.maximum(cnt, 1.0)
    v = _lrelu(_dotT(nf, wv1_ref[...]) + bv1_ref[...])
    v = _lrelu(_dotT(v, wv2_ref[...]) + bv2_ref[...])
    v = _lrelu(_dotT(v, wv3_ref[...]) + bv3_ref[...])
    v = _lrelu(_dotT(v, wv4_ref[...]) + bv4_ref[...])
    vout = _dotT(v, wv5_ref[...])
    xn = xn + vout * nf[:, 0:2]
    a = jnp.concatenate([nf, h_neigh], axis=1)
    a = _lrelu(_dotT(a, wn1_ref[...]) + bn1_ref[...])
    a = _lrelu(_dotT(a, wn2_ref[...]) + bn2_ref[...])
    a = _lrelu(_dotT(a, wn3_ref[...]) + bn3_ref[...])
    a = _lrelu(_dotT(a, wn4_ref[...]) + bn4_ref[...])
    a = _dotT(a, wn5_ref[...]) + bn5_ref[...]
    h = _dotT(a, wl_ref[...]) + bl_ref[...]
    out_ref[...] = jnp.concatenate([xn, h], axis=1)


def _post(nf, p0, p1, wts):
    blk = 1000
    grid = (_N // blk,)
    full = lambda shp: pl.BlockSpec(shp, lambda i: tuple(0 for _ in shp))
    w_specs = [full(w.shape) for w in wts]
    return pl.pallas_call(
        _post_body,
        grid=grid,
        in_specs=[
            pl.BlockSpec((blk, _IN), lambda i: (i, 0)),
            pl.BlockSpec((1, blk, _WROW), lambda i: (0, i, 0)),
            pl.BlockSpec((1, blk, _WROW), lambda i: (1, i, 0)),
            pl.BlockSpec((1, blk, _WROW), lambda i: (0, i, 0)),
            pl.BlockSpec((1, blk, _WROW), lambda i: (1, i, 0)),
        ] + w_specs,
        out_specs=pl.BlockSpec((blk, 2 + _OUT), lambda i: (i, 0)),
        out_shape=jax.ShapeDtypeStruct((_N, 2 + _OUT), _f32),
    )(nf, p0, p0, p1, p1, *wts)


def kernel(node_feat, feat, edge_index, edge_attr, params):
    w1, b1 = params["edge_mlp"][0]
    w1s = w1[:, :_IN]
    w1d = w1[:, _IN:2 * _IN]
    w1r = w1[:, 2 * _IN:2 * _IN + 1].T
    w1e = w1[:, 2 * _IN + 1:].astype(_bf16)
    b1r = b1.reshape(1, _H)

    def row(b):
        return b.reshape(1, -1)

    em = params["edge_mlp"]
    cm = params["coord_mlp"]
    edge_wts = [w1e, w1r]
    for i in (1, 2, 3, 4):
        edge_wts += [em[i][0].astype(_bf16), row(em[i][1])]
    for i in (0, 1, 2, 3):
        edge_wts += [cm[i][0].astype(_bf16), row(cm[i][1])]
    edge_wts.append(cm[4][0])

    vm = params["vel_mlp"]
    nm = params["node_mlp"]
    node_wts = []
    for i in (0, 1, 2, 3):
        node_wts += [vm[i][0], row(vm[i][1])]
    node_wts.append(vm[4][0])
    for i in range(5):
        node_wts += [nm[i][0], row(nm[i][1])]
    node_wts += [params["linh"][0], row(params["linh"][1])]

    src = edge_index[0]
    dst = edge_index[1]
    pad = _E_PAD - _E
    padc = (_TOTC_PAD - _TOTC) * _CHUNK
    src_p = jnp.concatenate([src, jnp.zeros((pad + padc,), jnp.int32)])
    dst_g = jnp.concatenate([dst, jnp.zeros((pad + padc,), jnp.int32)])
    dst_s = jnp.concatenate([dst, jnp.full((pad,), _N, jnp.int32)])
    srcc = src_p.reshape(_TOTC_PAD, _CHUNK)
    dstc = dst_g.reshape(_TOTC_PAD, _CHUNK)
    dstsc = dst_s.reshape(_TOTC, _CHUNK)

    tab_s, tab_d = _pre(node_feat, feat, w1s, w1d, b1r)
    g0 = _gather(tab_s, tab_d, srcc, dstc, 0, _TOTC_0, 96, 16)
    g1 = _gather(tab_s, tab_d, srcc, dstc, _TOTC_0, _TOTC_1, 24, 24)
    msg0 = _edge(g0, edge_attr, edge_wts, 0)
    msg1 = _edge(g1, edge_attr, edge_wts, _E_0 // 2048)
    p0 = _scatter(msg0, dstsc, 0)
    p1 = _scatter(msg1, dstsc, _TOTC_0)
    out = _post(node_feat, p0, p1, node_wts)
    return out
